# Initial kernel scaffold; baseline (speedup 1.0000x reference)
import jax
import jax.numpy as jnp
from jax.experimental import pallas as pl


def kernel(pc1, pc2, feature1, feature2, params):
    raise NotImplementedError("write your pallas kernel here")


# speedup vs baseline: 1.0258x; 1.0258x over previous
import functools
import jax
import jax.numpy as jnp
from jax import lax
from jax.experimental import pallas as pl
from jax.experimental.pallas import tpu as pltpu

_pallas_call = pl.pallas_call
_F32 = jnp.float32


def _bn_m(x):
    axes = tuple(i for i in range(x.ndim) if i != 1)
    m = x.mean(axes, keepdims=True)
    v = x.var(axes, keepdims=True)
    return (x - m) / jnp.sqrt(v + 1e-5)


def _knn_m(k, q, r):
    qs = jnp.sum(q ** 2, -1)
    rs = jnp.sum(r ** 2, -1)
    d = qs[:, :, None] + rs[:, None, :] - 2.0 * jnp.einsum('bnc,bmc->bnm', q, r)
    return lax.top_k(-d, k)[1]


def _group_m(feat, idx):
    return jax.vmap(lambda f, i: f[:, i])(feat, idx)


def _sa_m(pos, feat, ws, w2s, npoint, nsample, use_feat):
    B, _, N = pos.shape
    idx = jnp.arange(npoint) * (N // npoint)
    new_pos = pos[:, :, idx]
    kidx = _knn_m(nsample, new_pos.transpose(0, 2, 1), pos.transpose(0, 2, 1))
    pos_diff = _group_m(pos, kidx) - new_pos[:, :, :, None]
    if use_feat:
        x = jnp.concatenate([pos_diff, _group_m(feat, kidx)], 1)
    else:
        x = pos_diff
    for W in ws:
        x = jax.nn.relu(_bn_m(jnp.einsum('oc,bcns->bons', W, x)))
    x = x.max(-1)
    for W in w2s:
        x = jax.nn.relu(_bn_m(jnp.einsum('oc,bcn->bon', W, x)))
    return new_pos, x


def _dsq_m(q, r):
    qs = jnp.sum(q ** 2, -1)
    rs = jnp.sum(r ** 2, -1)
    return qs[:, :, None] + rs[:, None, :] - 2.0 * jnp.einsum('bnc,bmc->bnm', q, r)


def _topk_body(ns, dt_ref, out_ref, d_scr):
    d_scr[...] = dt_ref[0]
    n, nq = d_scr.shape

    def step(t, _):
        d = d_scr[...]
        cmin = jnp.min(d, axis=0, keepdims=True)
        iota = lax.broadcasted_iota(jnp.int32, (n, nq), 0)
        idx = jnp.min(jnp.where(d == cmin, iota, n), axis=0, keepdims=True)
        d_scr[...] = jnp.where(iota == idx, jnp.inf, d)
        out_ref[0, pl.ds(t, 1), :] = idx
        return 0

    lax.fori_loop(0, ns, step, 0)


def _topk_idx(dt, ns):
    B, N, Nq = dt.shape
    return _pallas_call(
        functools.partial(_topk_body, ns),
        grid=(B,),
        in_specs=[pl.BlockSpec((1, N, Nq), lambda b: (b, 0, 0))],
        out_specs=pl.BlockSpec((1, ns, Nq), lambda b: (b, 0, 0)),
        out_shape=jax.ShapeDtypeStruct((B, ns, Nq), jnp.int32),
        scratch_shapes=[pltpu.VMEM((N, Nq), _F32)],
    )(dt)


def _grows(table, idx):
    N, C = table.shape
    out = None
    for base in range(0, N, 8):
        chunk = table[base:base + 8]
        lidx = jnp.clip(idx - base, 0, 7)
        g = jnp.take_along_axis(chunk, lidx, axis=0)
        if out is None:
            out = g
        else:
            out = jnp.where(idx >= base, g, out)
    return out


def _corr_body(K, soft_ref, f1_ref, p1_ref, p2_ref,
               val_ref, feat_ref, int_ref, corr_ref):
    _, Nq, _ = soft_ref.shape
    soft = soft_ref[0]
    rmax = jnp.max(soft, axis=1, keepdims=True)
    iota = lax.broadcasted_iota(jnp.int32, (Nq, Nq), 1)
    fidx = jnp.min(jnp.where(soft == rmax, iota, Nq),
                   axis=1, keepdims=True)

    iota_c = lax.broadcasted_iota(jnp.int32, (Nq, 1), 0)
    cur = rmax
    pv, pi = [], []
    for _t in range(K):
        rm = jnp.max(cur, axis=0, keepdims=True)
        ti = jnp.min(jnp.where(cur == rm, iota_c, Nq), axis=0, keepdims=True)
        pv.append(rm)
        pi.append(ti)
        cur = jnp.where(iota_c == ti, -jnp.inf, cur)
    topv = jnp.concatenate(pv, axis=1)
    topi = jnp.concatenate(pi, axis=0)

    val_ref[0] = topv
    fidxb = jnp.broadcast_to(fidx, (Nq, 8))
    topi8 = jnp.broadcast_to(topi, (K, 8))
    scidx = _grows(fidxb, topi8)[:, :1]
    f1b = f1_ref[0]
    cgf = f1b.shape[1]
    feat_ref[0] = _grows(f1b, jnp.broadcast_to(topi, (K, cgf)))
    int_ref[0] = _grows(p1_ref[0], jnp.broadcast_to(topi, (K, 3)))
    corr_ref[0] = _grows(p2_ref[0], jnp.broadcast_to(scidx, (K, 3)))


def _corr_select(soft, f1t, p1t, p2t, K=64):
    B, Nq, _ = soft.shape
    C = f1t.shape[2]
    return _pallas_call(
        functools.partial(_corr_body, K),
        grid=(B,),
        in_specs=[
            pl.BlockSpec((1, Nq, Nq), lambda b: (b, 0, 0)),
            pl.BlockSpec((1, Nq, C), lambda b: (b, 0, 0)),
            pl.BlockSpec((1, Nq, 3), lambda b: (b, 0, 0)),
            pl.BlockSpec((1, Nq, 3), lambda b: (b, 0, 0)),
        ],
        out_specs=[
            pl.BlockSpec((1, 1, K), lambda b: (b, 0, 0)),
            pl.BlockSpec((1, K, C), lambda b: (b, 0, 0)),
            pl.BlockSpec((1, K, 3), lambda b: (b, 0, 0)),
            pl.BlockSpec((1, K, 3), lambda b: (b, 0, 0)),
        ],
        out_shape=[
            jax.ShapeDtypeStruct((B, 1, K), _F32),
            jax.ShapeDtypeStruct((B, K, C), _F32),
            jax.ShapeDtypeStruct((B, K, 3), _F32),
            jax.ShapeDtypeStruct((B, K, 3), _F32),
        ],
    )(soft, f1t, p1t, p2t)


def _fe_group_body(ns, K, *refs):
    p2_ref, f2_ref, corr_ref, feat1_ref, idxp_ref, idxf_ref, out_ref = refs
    pg = _grows(p2_ref[0], idxp_ref[0])
    pd = pg.reshape(ns, K, 3) - corr_ref[0].reshape(1, K, 3)
    cf2 = f2_ref.shape[2]
    fg = _grows(f2_ref[0], idxf_ref[0])
    c1 = feat1_ref.shape[2]
    f1r = jnp.broadcast_to(feat1_ref[0].reshape(1, K, c1), (ns, K, c1))
    out_ref[0] = jnp.concatenate([pd, fg.reshape(ns, K, cf2), f1r], axis=2)


def _fe_group(p2t, f2t, corr0, feat1, kidx):
    B, Nq, _ = p2t.shape
    _, ns, K = kidx.shape
    C2 = f2t.shape[2]
    C1 = feat1.shape[2]
    CT = 3 + C2 + C1
    m = ns * K
    idx_col = kidx.reshape(B, m, 1)
    idxp = jnp.broadcast_to(idx_col, (B, m, 3))
    idxf = jnp.broadcast_to(idx_col, (B, m, C2))
    return _pallas_call(
        functools.partial(_fe_group_body, ns, K),
        grid=(B,),
        in_specs=[
            pl.BlockSpec((1, Nq, 3), lambda b: (b, 0, 0)),
            pl.BlockSpec((1, Nq, C2), lambda b: (b, 0, 0)),
            pl.BlockSpec((1, K, 3), lambda b: (b, 0, 0)),
            pl.BlockSpec((1, K, C1), lambda b: (b, 0, 0)),
            pl.BlockSpec((1, m, 3), lambda b: (b, 0, 0)),
            pl.BlockSpec((1, m, C2), lambda b: (b, 0, 0)),
        ],
        out_specs=pl.BlockSpec((1, ns, K, CT), lambda b: (b, 0, 0, 0)),
        out_shape=jax.ShapeDtypeStruct((B, ns, K, CT), _F32),
    )(p2t, f2t, corr0, feat1, idxp, idxf)


def _fe_mlp_body(*refs):
    (x_ref, w0_ref, w1_ref, u0_ref, u1_ref, u2_ref, corr_ref, out_ref) = refs
    B, nsd, K, CT = x_ref.shape
    eps = 1e-5

    def bnrelu2(y, cnt):
        m = jnp.sum(y, axis=0, keepdims=True) / cnt
        v = jnp.maximum(jnp.sum(y * y, axis=0, keepdims=True) / cnt - m * m, 0.0)
        return jnp.maximum((y - m) / jnp.sqrt(v + eps), 0.0)

    mrows = B * nsd * K
    x = x_ref[...].reshape(mrows, CT)
    y = jnp.dot(x, w0_ref[...], preferred_element_type=_F32)
    x = bnrelu2(y, float(mrows))
    y = jnp.dot(x, w1_ref[...], preferred_element_type=_F32)
    x = bnrelu2(y, float(mrows))
    c2 = x.shape[1]
    x = jnp.max(x.reshape(B, nsd, K, c2), axis=1).reshape(B * K, c2)
    for uref in (u0_ref, u1_ref, u2_ref):
        y = jnp.dot(x, uref[...], preferred_element_type=_F32)
        x = bnrelu2(y, float(B * K))
    out_ref[...] = corr_ref[...] + x.reshape(B, K, 3)


def _fe_mlp(x, fe_t, fe2_t, corr0):
    B, ns, K, CT = x.shape
    full = lambda a: pl.BlockSpec(a.shape, lambda: tuple(0 for _ in a.shape))
    args = [x, fe_t[0], fe_t[1], fe2_t[0], fe2_t[1], fe2_t[2], corr0]
    return _pallas_call(
        _fe_mlp_body,
        in_specs=[full(a) for a in args],
        out_specs=pl.BlockSpec((B, K, 3), lambda: (0, 0, 0)),
        out_shape=jax.ShapeDtypeStruct((B, K, 3), _F32),
    )(*args)


def kernel(pc1, pc2, feature1, feature2, params):
    l1_pc1, l1_f1 = _sa_m(pc1, feature1, params['sa1_w'], params['sa1_w2'], 512, 32, False)
    l2_pc1, l2_f1 = _sa_m(l1_pc1, l1_f1, params['sa2_w'], params['sa2_w2'], 256, 64, True)
    l1_pc2, l1_f2 = _sa_m(pc2, feature2, params['sa1_w'], params['sa1_w2'], 512, 32, False)
    l2_pc2, l2_f2 = _sa_m(l1_pc2, l1_f2, params['sa2_w'], params['sa2_w2'], 256, 64, True)

    def convblock(x):
        for W in params['conv_w']:
            x = jax.nn.relu(_bn_m(jnp.einsum('oc,bcn->bon', W, x)))
        return x

    f1n = convblock(l2_f1)
    f2n = convblock(l2_f2)
    inner = -2.0 * jnp.einsum('bcn,bcm->bnm', f1n, f2n)
    ssq = jnp.sum(f1n ** 2, 1)
    tsq = jnp.sum(f2n ** 2, 1)
    attn = -tsq[:, None, :] - inner - ssq[:, :, None]
    f2_attn = jax.nn.softmax(attn, -1)

    p1t = l2_pc1.transpose(0, 2, 1)
    p2t = l2_pc2.transpose(0, 2, 1)
    f1t = l2_f1.transpose(0, 2, 1)
    f2t = l2_f2.transpose(0, 2, 1)
    topv, cfeat, interest, corr0 = _corr_select(f2_attn, f1t, p1t, p2t, 64)
    feat1 = jnp.concatenate([cfeat, topv.transpose(0, 2, 1)], axis=2)

    d = _dsq_m(corr0, p2t)
    kidx = _topk_idx(d.transpose(0, 2, 1), 16)
    fe_t = [w.T for w in params['fe_w']]
    fe2_t = [w.T for w in params['fe_w2']]
    xg = _fe_group(p2t, f2t, corr0, feat1, kidx)
    corr = _fe_mlp(xg, fe_t, fe2_t, corr0)

    return (interest.transpose(0, 2, 1), corr.transpose(0, 2, 1),
            l2_pc1, l2_pc2, f2_attn)


# speedup vs baseline: 1.3987x; 1.3636x over previous
import functools
import jax
import jax.numpy as jnp
from jax import lax
from jax.experimental import pallas as pl
from jax.experimental.pallas import tpu as pltpu

_pallas_call = pl.pallas_call
_F32 = jnp.float32


def _bn_m(x):
    axes = tuple(i for i in range(x.ndim) if i != 1)
    m = x.mean(axes, keepdims=True)
    v = x.var(axes, keepdims=True)
    return (x - m) / jnp.sqrt(v + 1e-5)


def _group_m(feat, idx):
    return jax.vmap(lambda f, i: f[:, i])(feat, idx)


def _sa_m(pos, feat, ws, w2s, npoint, nsample, use_feat):
    B, _, N = pos.shape
    idx = jnp.arange(npoint) * (N // npoint)
    new_pos = pos[:, :, idx]
    d = _dsq_m(new_pos.transpose(0, 2, 1), pos.transpose(0, 2, 1))
    kidx = _topk_idx(d.transpose(0, 2, 1), nsample).transpose(0, 2, 1)
    pos_diff = _group_m(pos, kidx) - new_pos[:, :, :, None]
    if use_feat:
        x = jnp.concatenate([pos_diff, _group_m(feat, kidx)], 1)
    else:
        x = pos_diff
    for W in ws:
        x = jax.nn.relu(_bn_m(jnp.einsum('oc,bcns->bons', W, x)))
    x = x.max(-1)
    for W in w2s:
        x = jax.nn.relu(_bn_m(jnp.einsum('oc,bcn->bon', W, x)))
    return new_pos, x


def _dsq_m(q, r):
    qs = jnp.sum(q ** 2, -1)
    rs = jnp.sum(r ** 2, -1)
    return qs[:, :, None] + rs[:, None, :] - 2.0 * jnp.einsum('bnc,bmc->bnm', q, r)


def _topk_body(ns, dt_ref, out_ref, d_scr):
    d_scr[...] = dt_ref[0]
    n, nq = d_scr.shape

    def step(t, _):
        d = d_scr[...]
        cmin = jnp.min(d, axis=0, keepdims=True)
        iota = lax.broadcasted_iota(jnp.int32, (n, nq), 0)
        idx = jnp.min(jnp.where(d == cmin, iota, n), axis=0, keepdims=True)
        d_scr[...] = jnp.where(iota == idx, jnp.inf, d)
        out_ref[0, pl.ds(t, 1), :] = idx
        return 0

    lax.fori_loop(0, ns, step, 0)


def _topk_idx(dt, ns):
    B, N, Nq = dt.shape
    return _pallas_call(
        functools.partial(_topk_body, ns),
        grid=(B,),
        in_specs=[pl.BlockSpec((1, N, Nq), lambda b: (b, 0, 0))],
        out_specs=pl.BlockSpec((1, ns, Nq), lambda b: (b, 0, 0)),
        out_shape=jax.ShapeDtypeStruct((B, ns, Nq), jnp.int32),
        scratch_shapes=[pltpu.VMEM((N, Nq), _F32)],
    )(dt)


def _grows(table, idx):
    N, C = table.shape
    out = None
    for base in range(0, N, 8):
        chunk = table[base:base + 8]
        lidx = jnp.clip(idx - base, 0, 7)
        g = jnp.take_along_axis(chunk, lidx, axis=0)
        if out is None:
            out = g
        else:
            out = jnp.where(idx >= base, g, out)
    return out


def _corr_body(K, soft_ref, f1_ref, p1_ref, p2_ref,
               val_ref, feat_ref, int_ref, corr_ref):
    _, Nq, _ = soft_ref.shape
    soft = soft_ref[0]
    rmax = jnp.max(soft, axis=1, keepdims=True)
    iota = lax.broadcasted_iota(jnp.int32, (Nq, Nq), 1)
    fidx = jnp.min(jnp.where(soft == rmax, iota, Nq),
                   axis=1, keepdims=True)

    iota_c = lax.broadcasted_iota(jnp.int32, (Nq, 1), 0)
    cur = rmax
    pv, pi = [], []
    for _t in range(K):
        rm = jnp.max(cur, axis=0, keepdims=True)
        ti = jnp.min(jnp.where(cur == rm, iota_c, Nq), axis=0, keepdims=True)
        pv.append(rm)
        pi.append(ti)
        cur = jnp.where(iota_c == ti, -jnp.inf, cur)
    topv = jnp.concatenate(pv, axis=1)
    topi = jnp.concatenate(pi, axis=0)

    val_ref[0] = topv
    fidxb = jnp.broadcast_to(fidx, (Nq, 8))
    topi8 = jnp.broadcast_to(topi, (K, 8))
    scidx = _grows(fidxb, topi8)[:, :1]
    f1b = f1_ref[0]
    cgf = f1b.shape[1]
    feat_ref[0] = _grows(f1b, jnp.broadcast_to(topi, (K, cgf)))
    int_ref[0] = _grows(p1_ref[0], jnp.broadcast_to(topi, (K, 3)))
    corr_ref[0] = _grows(p2_ref[0], jnp.broadcast_to(scidx, (K, 3)))


def _corr_select(soft, f1t, p1t, p2t, K=64):
    B, Nq, _ = soft.shape
    C = f1t.shape[2]
    return _pallas_call(
        functools.partial(_corr_body, K),
        grid=(B,),
        in_specs=[
            pl.BlockSpec((1, Nq, Nq), lambda b: (b, 0, 0)),
            pl.BlockSpec((1, Nq, C), lambda b: (b, 0, 0)),
            pl.BlockSpec((1, Nq, 3), lambda b: (b, 0, 0)),
            pl.BlockSpec((1, Nq, 3), lambda b: (b, 0, 0)),
        ],
        out_specs=[
            pl.BlockSpec((1, 1, K), lambda b: (b, 0, 0)),
            pl.BlockSpec((1, K, C), lambda b: (b, 0, 0)),
            pl.BlockSpec((1, K, 3), lambda b: (b, 0, 0)),
            pl.BlockSpec((1, K, 3), lambda b: (b, 0, 0)),
        ],
        out_shape=[
            jax.ShapeDtypeStruct((B, 1, K), _F32),
            jax.ShapeDtypeStruct((B, K, C), _F32),
            jax.ShapeDtypeStruct((B, K, 3), _F32),
            jax.ShapeDtypeStruct((B, K, 3), _F32),
        ],
    )(soft, f1t, p1t, p2t)


def _fe_group_body(ns, K, *refs):
    p2_ref, f2_ref, corr_ref, feat1_ref, idxp_ref, idxf_ref, out_ref = refs
    pg = _grows(p2_ref[0], idxp_ref[0])
    pd = pg.reshape(ns, K, 3) - corr_ref[0].reshape(1, K, 3)
    cf2 = f2_ref.shape[2]
    fg = _grows(f2_ref[0], idxf_ref[0])
    c1 = feat1_ref.shape[2]
    f1r = jnp.broadcast_to(feat1_ref[0].reshape(1, K, c1), (ns, K, c1))
    out_ref[0] = jnp.concatenate([pd, fg.reshape(ns, K, cf2), f1r], axis=2)


def _fe_group(p2t, f2t, corr0, feat1, kidx):
    B, Nq, _ = p2t.shape
    _, ns, K = kidx.shape
    C2 = f2t.shape[2]
    C1 = feat1.shape[2]
    CT = 3 + C2 + C1
    m = ns * K
    idx_col = kidx.reshape(B, m, 1)
    idxp = jnp.broadcast_to(idx_col, (B, m, 3))
    idxf = jnp.broadcast_to(idx_col, (B, m, C2))
    return _pallas_call(
        functools.partial(_fe_group_body, ns, K),
        grid=(B,),
        in_specs=[
            pl.BlockSpec((1, Nq, 3), lambda b: (b, 0, 0)),
            pl.BlockSpec((1, Nq, C2), lambda b: (b, 0, 0)),
            pl.BlockSpec((1, K, 3), lambda b: (b, 0, 0)),
            pl.BlockSpec((1, K, C1), lambda b: (b, 0, 0)),
            pl.BlockSpec((1, m, 3), lambda b: (b, 0, 0)),
            pl.BlockSpec((1, m, C2), lambda b: (b, 0, 0)),
        ],
        out_specs=pl.BlockSpec((1, ns, K, CT), lambda b: (b, 0, 0, 0)),
        out_shape=jax.ShapeDtypeStruct((B, ns, K, CT), _F32),
    )(p2t, f2t, corr0, feat1, idxp, idxf)


def _fe_mlp_body(*refs):
    (x_ref, w0_ref, w1_ref, u0_ref, u1_ref, u2_ref, corr_ref, out_ref) = refs
    B, nsd, K, CT = x_ref.shape
    eps = 1e-5

    def bnrelu2(y, cnt):
        m = jnp.sum(y, axis=0, keepdims=True) / cnt
        v = jnp.maximum(jnp.sum(y * y, axis=0, keepdims=True) / cnt - m * m, 0.0)
        return jnp.maximum((y - m) / jnp.sqrt(v + eps), 0.0)

    mrows = B * nsd * K
    x = x_ref[...].reshape(mrows, CT)
    y = jnp.dot(x, w0_ref[...], preferred_element_type=_F32)
    x = bnrelu2(y, float(mrows))
    y = jnp.dot(x, w1_ref[...], preferred_element_type=_F32)
    x = bnrelu2(y, float(mrows))
    c2 = x.shape[1]
    x = jnp.max(x.reshape(B, nsd, K, c2), axis=1).reshape(B * K, c2)
    for uref in (u0_ref, u1_ref, u2_ref):
        y = jnp.dot(x, uref[...], preferred_element_type=_F32)
        x = bnrelu2(y, float(B * K))
    out_ref[...] = corr_ref[...] + x.reshape(B, K, 3)


def _fe_mlp(x, fe_t, fe2_t, corr0):
    B, ns, K, CT = x.shape
    full = lambda a: pl.BlockSpec(a.shape, lambda: tuple(0 for _ in a.shape))
    args = [x, fe_t[0], fe_t[1], fe2_t[0], fe2_t[1], fe2_t[2], corr0]
    return _pallas_call(
        _fe_mlp_body,
        in_specs=[full(a) for a in args],
        out_specs=pl.BlockSpec((B, K, 3), lambda: (0, 0, 0)),
        out_shape=jax.ShapeDtypeStruct((B, K, 3), _F32),
    )(*args)


def kernel(pc1, pc2, feature1, feature2, params):
    l1_pc1, l1_f1 = _sa_m(pc1, feature1, params['sa1_w'], params['sa1_w2'], 512, 32, False)
    l2_pc1, l2_f1 = _sa_m(l1_pc1, l1_f1, params['sa2_w'], params['sa2_w2'], 256, 64, True)
    l1_pc2, l1_f2 = _sa_m(pc2, feature2, params['sa1_w'], params['sa1_w2'], 512, 32, False)
    l2_pc2, l2_f2 = _sa_m(l1_pc2, l1_f2, params['sa2_w'], params['sa2_w2'], 256, 64, True)

    def convblock(x):
        for W in params['conv_w']:
            x = jax.nn.relu(_bn_m(jnp.einsum('oc,bcn->bon', W, x)))
        return x

    f1n = convblock(l2_f1)
    f2n = convblock(l2_f2)
    inner = -2.0 * jnp.einsum('bcn,bcm->bnm', f1n, f2n)
    ssq = jnp.sum(f1n ** 2, 1)
    tsq = jnp.sum(f2n ** 2, 1)
    attn = -tsq[:, None, :] - inner - ssq[:, :, None]
    f2_attn = jax.nn.softmax(attn, -1)

    p1t = l2_pc1.transpose(0, 2, 1)
    p2t = l2_pc2.transpose(0, 2, 1)
    f1t = l2_f1.transpose(0, 2, 1)
    f2t = l2_f2.transpose(0, 2, 1)
    topv, cfeat, interest, corr0 = _corr_select(f2_attn, f1t, p1t, p2t, 64)
    feat1 = jnp.concatenate([cfeat, topv.transpose(0, 2, 1)], axis=2)

    d = _dsq_m(corr0, p2t)
    kidx = _topk_idx(d.transpose(0, 2, 1), 16)
    fe_t = [w.T for w in params['fe_w']]
    fe2_t = [w.T for w in params['fe_w2']]
    xg = _fe_group(p2t, f2t, corr0, feat1, kidx)
    corr = _fe_mlp(xg, fe_t, fe2_t, corr0)

    return (interest.transpose(0, 2, 1), corr.transpose(0, 2, 1),
            l2_pc1, l2_pc2, f2_attn)
